# baseline (speedup 1.0000x reference)
import functools

import jax
import jax.numpy as jnp
from jax import lax
from jax.experimental import pallas as pl
from jax.experimental.pallas import tpu as pltpu
from jax.experimental.pallas import tpu_sc as plsc

_D = 64
_B = 4096
_L = 200
_PAD = 1
_NW = 32
_RPW = _B // _NW
_NCH = 13
_IDXCAP = 224


def _sc_pool(text_flat, mask_flat, emb):
    mesh = plsc.VectorSubcoreMesh(core_axis_name="c", subcore_axis_name="s")

    @functools.partial(
        pl.kernel,
        mesh=mesh,
        compiler_params=pltpu.CompilerParams(
            use_tc_tiling_on_sc=False, needs_layout_passes=False
        ),
        out_type=jax.ShapeDtypeStruct((_B, _D), jnp.float32),
        scratch_types=[
            pltpu.VMEM((_RPW * _L,), jnp.int32),
            pltpu.VMEM((_RPW * _L,), jnp.int32),
            pltpu.VMEM((_IDXCAP,), jnp.int32),
            pltpu.VMEM((_NCH * 16, _D), jnp.float32),
            pltpu.VMEM((_RPW, _D), jnp.float32),
            pltpu.SemaphoreType.DMA,
        ],
    )
    def k(text_hbm, mask_hbm, emb_hbm, sums_hbm,
          text_v, mask_v, idx_v, rows_v, sums_v, sem):
        wid = lax.axis_index("s") * 2 + lax.axis_index("c")
        base = wid * _RPW
        pltpu.sync_copy(text_hbm.at[pl.ds(base * _L, _RPW * _L)], text_v)
        pltpu.sync_copy(mask_hbm.at[pl.ds(base * _L, _RPW * _L)], mask_v)

        lane = lax.iota(jnp.int32, 16)
        pad_vec = jnp.full((16,), _PAD, jnp.int32)
        zeros = jnp.zeros((16,), jnp.float32)

        def row_body(i, carry):
            rowoff = i * _L
            for kk in range(_IDXCAP // 16):
                idx_v[pl.ds(kk * 16, 16)] = pad_vec
            off_vec = jnp.zeros((16,), jnp.int32)
            for kk in range(_NCH):
                start = kk * 16 if kk < _NCH - 1 else _L - 16
                t = text_v[pl.ds(rowoff + start, 16)]
                m = mask_v[pl.ds(rowoff + start, 16)]
                if kk == _NCH - 1:
                    m = jnp.where(lane >= 16 * _NCH - _L, m, 0)
                mb = m != 0
                mi = jnp.where(mb, 1, 0)
                pos = plsc.cumsum(mi) + off_vec - 1
                plsc.store_scatter(idx_v, [pos], t, mask=mb)
                off_vec = off_vec + plsc.all_reduce_population_count(mb)

            pltpu.make_async_copy(
                emb_hbm.at[idx_v.at[pl.ds(0, _NCH * 16)]],
                rows_v,
                sem,
            ).start()
            pltpu.make_async_copy(
                emb_hbm.at[idx_v.at[pl.ds(0, _NCH * 16)]],
                rows_v,
                sem,
            ).wait()

            for v in range(4):
                sums_v[i, pl.ds(v * 16, 16)] = zeros
            for c in range(_NCH):
                for v in range(4):
                    av = sums_v[i, pl.ds(v * 16, 16)]
                    for rr in range(16):
                        av = av + rows_v[c * 16 + rr, pl.ds(v * 16, 16)]
                    sums_v[i, pl.ds(v * 16, 16)] = av
            return carry

        lax.fori_loop(0, _RPW, row_body, 0)
        pltpu.sync_copy(sums_v, sums_hbm.at[pl.ds(base, _RPW)])

    return k(text_flat, mask_flat, emb)


def _tc_body(sums_ref, mask_ref, w_ref, b_ref, out_ref):
    cnt = jnp.sum(mask_ref[...].astype(jnp.float32), axis=1, keepdims=True)
    sent = sums_ref[...] / cnt
    out_ref[...] = (
        jnp.dot(sent, w_ref[...], preferred_element_type=jnp.float32) + b_ref[...]
    )


def kernel(text, mask, emb, W, b):
    sums = _sc_pool(text.reshape(-1), mask.reshape(-1), emb)
    return pl.pallas_call(
        _tc_body,
        out_shape=jax.ShapeDtypeStruct((_B, 2), jnp.float32),
    )(sums, mask, W, b.reshape(1, 2))
